# baseline (speedup 1.0000x reference)
import jax
import jax.numpy as jnp
from jax.experimental import pallas as pl
from jax.experimental.pallas import tpu as pltpu

_BM = 1024


def _router_block(x_ref, w_hbm, b_hbm, out_ref, w_buf, b_buf, sem):
    i = pl.program_id(0)

    @pl.when(i == 0)
    def _():
        pltpu.make_async_copy(w_hbm, w_buf, sem.at[0]).start()
        pltpu.make_async_copy(b_hbm, b_buf, sem.at[1]).start()
        pltpu.make_async_copy(w_hbm, w_buf, sem.at[0]).wait()
        pltpu.make_async_copy(b_hbm, b_buf, sem.at[1]).wait()

    logits = jax.lax.dot_general(
        x_ref[...], w_buf[...],
        dimension_numbers=(((1,), (1,)), ((), ())),
        preferred_element_type=jnp.float32,
    )
    logits = logits + b_buf[...]
    m = jnp.max(logits, axis=-1, keepdims=True)
    e = jnp.exp(logits - m)
    out_ref[...] = e / jnp.sum(e, axis=-1, keepdims=True)


def kernel(x, W, b):
    n_tokens, d_model = x.shape
    n_experts = W.shape[0]
    grid = (n_tokens // _BM,)
    return pl.pallas_call(
        _router_block,
        grid=grid,
        in_specs=[
            pl.BlockSpec((_BM, d_model), lambda i: (i, 0)),
            pl.BlockSpec(memory_space=pltpu.MemorySpace.HBM),
            pl.BlockSpec(memory_space=pltpu.MemorySpace.HBM),
        ],
        out_specs=pl.BlockSpec((_BM, n_experts), lambda i: (i, 0)),
        out_shape=jax.ShapeDtypeStruct((n_tokens, n_experts), jnp.float32),
        scratch_shapes=[
            pltpu.VMEM((n_experts, d_model), jnp.float32),
            pltpu.VMEM((1, n_experts), jnp.float32),
            pltpu.SemaphoreType.DMA((2,)),
        ],
        compiler_params=pltpu.CompilerParams(
            dimension_semantics=("arbitrary",),
        ),
    )(x, W, b.reshape(1, n_experts))
